# baseline (speedup 1.0000x reference)
import functools
import math

import jax
import jax.numpy as jnp
from jax import lax
from jax.experimental import pallas as pl
from jax.experimental.pallas import tpu as pltpu
from jax.experimental.pallas import tpu_sc as plsc

N = 10000
E = 320000
NP = 10240
D = 128
ROWB = 256

NC = 2
NS = 16
NW = NC * NS
EPW = 10240
E2 = NW * EPW
CHS = 80
NCHS = EPW // CHS
FANS = 4
CHR = 128
NCHR = EPW // CHR
NPS = NP // NS


def _nblocks():
    return NP // ROWB


def _tca_body(h_ref, w_ref, d0_ref, d1_ref, a_ref, xw_ref, y2_ref, ad_ref, dis2_ref):
    a = a_ref[...]
    deg = a * (d0_ref[...] + d1_ref[...]) + 1.0
    dis2 = 1.0 / deg
    dis = jnp.sqrt(dis2)
    ad = a * dis
    xw = jnp.dot(h_ref[...], w_ref[...], preferred_element_type=jnp.float32)
    xw_ref[...] = xw
    y2_ref[...] = xw * ad[:, None]
    ad_ref[...] = ad
    dis2_ref[...] = dis2


def _tca(h, W, degsum2, a):
    grid = (_nblocks(),)
    rb = pl.BlockSpec((ROWB, D), lambda i: (i, 0))
    vb = pl.BlockSpec((ROWB,), lambda i: (i,))
    wb = pl.BlockSpec((D, D), lambda i: (0, 0))
    return pl.pallas_call(
        _tca_body,
        grid=grid,
        in_specs=[rb, wb, vb, vb, vb],
        out_specs=[rb, rb, vb, vb],
        out_shape=[
            jax.ShapeDtypeStruct((NP, D), jnp.float32),
            jax.ShapeDtypeStruct((NP, D), jnp.float32),
            jax.ShapeDtypeStruct((NP,), jnp.float32),
            jax.ShapeDtypeStruct((NP,), jnp.float32),
        ],
    )(h, W, degsum2[0], degsum2[1], a)


def _tcb_body(r0_ref, r1_ref, xw_ref, ad_ref, dis2_ref, b_ref, ws_ref, hh_ref, xws_ref, z_ref):
    ad = ad_ref[...]
    hh = jnp.maximum(
        ad[:, None] * (r0_ref[...] + r1_ref[...])
        + xw_ref[...] * dis2_ref[...][:, None] + b_ref[...][None, :],
        0.0,
    )
    hh_ref[...] = hh
    xws = jnp.sum(hh * ws_ref[...][None, :], axis=1)
    xws_ref[...] = xws
    z_ref[...] = xws * ad


def _tcb(ragg2, xw, ad, dis2, b, Ws):
    grid = (_nblocks(),)
    rb = pl.BlockSpec((ROWB, D), lambda i: (i, 0))
    vb = pl.BlockSpec((ROWB,), lambda i: (i,))
    db = pl.BlockSpec((D,), lambda i: (0,))
    return pl.pallas_call(
        _tcb_body,
        grid=grid,
        in_specs=[rb, rb, rb, vb, vb, db, db],
        out_specs=[rb, vb, vb],
        out_shape=[
            jax.ShapeDtypeStruct((NP, D), jnp.float32),
            jax.ShapeDtypeStruct((NP,), jnp.float32),
            jax.ShapeDtypeStruct((NP,), jnp.float32),
        ],
    )(ragg2[0], ragg2[1], xw, ad, dis2, b, Ws[:, 0])


def _tcc_body(k, hh_ref, s0_ref, s1_ref, xws_ref, ad_ref, dis2_ref, a_ref, bs_ref,
              hnext_ref, anext_ref, ro_ref):
    ad = ad_ref[...]
    a = a_ref[...]
    score = ad * (s0_ref[...] + s1_ref[...]) + xws_ref[...] * dis2_ref[...] + bs_ref[0]
    bits = lax.bitcast_convert_type(score, jnp.uint32)
    key = jnp.where(score >= 0, bits | jnp.uint32(0x80000000), ~bits)
    key = jnp.where(a > 0, key, jnp.uint32(0))

    def cnt_ge(t):
        return jnp.sum((key >= t).astype(jnp.int32))

    t = jnp.uint32(0)
    for bit in range(31, -1, -1):
        cand = t | jnp.uint32(1 << bit)
        t = jnp.where(cnt_ge(cand) >= k, cand, t)
    need = k - jnp.sum((key > t).astype(jnp.int32))
    idx = lax.broadcasted_iota(jnp.int32, (NP,), 0)
    iseq = key == t
    u = jnp.int32(0)
    for bit in range(14, -1, -1):
        cand = u + jnp.int32(1 << bit)
        c = jnp.sum((iseq & (idx < cand)).astype(jnp.int32))
        u = jnp.where(c <= need, cand, u)
    kept = (key > t) | (iseq & (idx < u))
    keptf = kept.astype(jnp.float32)
    g = keptf * jnp.tanh(score)
    hn = hh_ref[...] * g[:, None]
    hnext_ref[...] = hn
    anext_ref[...] = keptf
    mx = jnp.max(jnp.where(keptf[:, None] > 0, hn, -jnp.inf), axis=0)
    mn = jnp.sum(hn * keptf[:, None], axis=0) * (1.0 / k)
    ro_ref[0, :D] = mx
    ro_ref[0, D:] = mn


def _tcc(k, hh, sagg2, xws, ad, dis2, a, bs):
    return pl.pallas_call(
        functools.partial(_tcc_body, k),
        out_shape=[
            jax.ShapeDtypeStruct((NP, D), jnp.float32),
            jax.ShapeDtypeStruct((NP,), jnp.float32),
            jax.ShapeDtypeStruct((1, 2 * D), jnp.float32),
        ],
    )(hh, sagg2[0], sagg2[1], xws, ad, dis2, a, bs)


def _tcd_body(s_ref, l1w_ref, l1b_ref, l2w_ref, l2b_ref, l3w_ref, l3b_ref, out_ref):
    s = s_ref[...]
    s = jnp.maximum(jnp.dot(s, l1w_ref[...], preferred_element_type=jnp.float32) + l1b_ref[...][None, :], 0.0)
    s = jnp.maximum(jnp.dot(s, l2w_ref[...], preferred_element_type=jnp.float32) + l2b_ref[...][None, :], 0.0)
    out_ref[...] = jnp.dot(s, l3w_ref[...], preferred_element_type=jnp.float32) + l3b_ref[...][None, :]


def _tcd(s, L1w, L1b, L2w, L2b, L3w, L3b):
    return pl.pallas_call(
        _tcd_body,
        out_shape=jax.ShapeDtypeStruct((1, 10), jnp.float32),
    )(s, L1w, L1b, L2w, L2b, L3w, L3b)


_SC_MESH = plsc.VectorSubcoreMesh(core_axis_name="c", subcore_axis_name="s")


@functools.partial(
    pl.kernel,
    out_type=jax.ShapeDtypeStruct((2, NP), jnp.float32),
    mesh=_SC_MESH,
    scratch_types=[
        pltpu.VMEM((NCHS, CHS), jnp.int32),
        pltpu.VMEM((NCHS, CHS), jnp.int32),
        pltpu.VMEM((FANS * CHS,), jnp.float32),
        pltpu.VMEM_SHARED((NP,), jnp.float32),
        [pltpu.SemaphoreType.DMA] * FANS,
    ],
)
def _sc_seg_scalar(vals_hbm, src_hbm, dst_hbm, zvec_hbm, out_hbm,
                   src_v, dst_v, buf_v, acc_sh, sems):
    cid = lax.axis_index("c")
    sid = lax.axis_index("s")
    wid = sid * NC + cid
    pltpu.sync_copy(zvec_hbm, acc_sh.at[pl.ds(sid * NPS, NPS)])
    pltpu.sync_copy(src_hbm.at[wid], src_v)
    pltpu.sync_copy(dst_hbm.at[wid], dst_v)
    plsc.subcore_barrier()

    def body(jj, carry):
        base = jj * FANS
        cps = [
            pltpu.async_copy(vals_hbm.at[src_v.at[base + b]],
                             buf_v.at[pl.ds(b * CHS, CHS)], sems[b])
            for b in range(FANS)
        ]
        for b in range(FANS):
            cps[b].wait()
            pltpu.sync_copy(buf_v.at[pl.ds(b * CHS, CHS)],
                            acc_sh.at[dst_v.at[base + b]], add=True)
        return carry

    lax.fori_loop(0, NCHS // FANS, body, 0)
    plsc.subcore_barrier()
    pltpu.sync_copy(acc_sh.at[pl.ds(sid * NPS, NPS)],
                    out_hbm.at[cid, pl.ds(sid * NPS, NPS)])


@functools.partial(
    pl.kernel,
    out_type=jax.ShapeDtypeStruct((2, NP, D), jnp.float32),
    mesh=_SC_MESH,
    scratch_types=[
        pltpu.VMEM((NCHR, CHR), jnp.int32),
        pltpu.VMEM((NCHR, CHR), jnp.int32),
        pltpu.VMEM((CHR, D), jnp.float32),
        pltpu.VMEM_SHARED((NP, D), jnp.float32),
        pltpu.SemaphoreType.DMA,
    ],
)
def _sc_seg_rows(y_hbm, src_hbm, dst_hbm, zrows_hbm, out_hbm,
                 src_v, dst_v, rows_v, acc_sh, sem):
    cid = lax.axis_index("c")
    sid = lax.axis_index("s")
    wid = sid * NC + cid
    pltpu.sync_copy(zrows_hbm, acc_sh.at[pl.ds(sid * NPS, NPS)])
    pltpu.sync_copy(src_hbm.at[wid], src_v)
    pltpu.sync_copy(dst_hbm.at[wid], dst_v)
    plsc.subcore_barrier()

    def body(j, carry):
        pltpu.async_copy(y_hbm.at[src_v.at[j]], rows_v, sem).wait()
        pltpu.sync_copy(rows_v, acc_sh.at[dst_v.at[j]], add=True)
        return carry

    lax.fori_loop(0, NCHR, body, 0)
    plsc.subcore_barrier()
    pltpu.sync_copy(acc_sh.at[pl.ds(sid * NPS, NPS)],
                    out_hbm.at[cid, pl.ds(sid * NPS, NPS)])


def kernel(x, edge_index, batch, W1, b1, Ws1, bs1, W2, b2, Ws2, bs2, W3, b3,
           Ws3, bs3, L1w, L1b, L2w, L2b, L3w, L3b):
    pad = jnp.full((E2 - E,), NP - 1, dtype=edge_index.dtype)
    srcp = jnp.concatenate([edge_index[0], pad])
    dstp = jnp.concatenate([edge_index[1], pad])
    src3 = jnp.reshape(srcp, (NW, NCHS, CHS))
    dst3 = jnp.reshape(dstp, (NW, NCHS, CHS))
    src2r = jnp.reshape(srcp, (NW, NCHR, CHR))
    dst2r = jnp.reshape(dstp, (NW, NCHR, CHR))
    zvec = jnp.zeros((NPS,), jnp.float32)
    zrows = jnp.zeros((NPS, D), jnp.float32)
    h = jnp.pad(x, ((0, NP - N), (0, 0)))
    a = jnp.pad(jnp.ones((N,), jnp.float32), (0, NP - N))

    ks = []
    kk = N
    for _ in range(3):
        kk = int(math.ceil(0.8 * kk))
        ks.append(kk)

    params = [(W1, b1, Ws1, bs1), (W2, b2, Ws2, bs2), (W3, b3, Ws3, bs3)]
    readouts = []
    for r in range(3):
        W, b, Wsc, bsc = params[r]
        k = ks[r]
        degsum2 = _sc_seg_scalar(a, src3, dst3, zvec)
        xw, y, ad, dis2 = _tca(h, W, degsum2, a)
        ragg2 = _sc_seg_rows(y, src2r, dst2r, zrows)
        hh, xws, z = _tcb(ragg2, xw, ad, dis2, b, Wsc)
        sagg2 = _sc_seg_scalar(z, src3, dst3, zvec)
        h, a, ro = _tcc(k, hh, sagg2, xws, ad, dis2, a, bsc)
        readouts.append(ro)

    s = readouts[0] + readouts[1] + readouts[2]
    return _tcd(s, L1w, L1b, L2w, L2b, L3w, L3b)


# speedup vs baseline: 2.1360x; 2.1360x over previous
import functools
import math

import jax
import jax.numpy as jnp
from jax import lax
from jax.experimental import pallas as pl
from jax.experimental.pallas import tpu as pltpu
from jax.experimental.pallas import tpu_sc as plsc

N = 10000
E = 320000
NP = 10240
D = 128
ROWB = 256

NC = 2
NS = 16
NW = NC * NS
EPW = 10240
E2 = NW * EPW
CHS = 80
NCHS = EPW // CHS
FANS = 4
CHR = 128
NCHR = EPW // CHR
NPS = NP // NS


def _nblocks():
    return NP // ROWB


def _tca_body(h_ref, w_ref, d0_ref, d1_ref, a_ref, xw_ref, y2_ref, ad_ref, dis2_ref):
    a = a_ref[...]
    deg = a * (d0_ref[...] + d1_ref[...]) + 1.0
    dis2 = 1.0 / deg
    dis = jnp.sqrt(dis2)
    ad = a * dis
    xw = jnp.dot(h_ref[...], w_ref[...], preferred_element_type=jnp.float32)
    xw_ref[...] = xw
    y2_ref[...] = xw * ad[:, None]
    ad_ref[...] = ad
    dis2_ref[...] = dis2


def _tca(h, W, degsum2, a):
    grid = (_nblocks(),)
    rb = pl.BlockSpec((ROWB, D), lambda i: (i, 0))
    vb = pl.BlockSpec((ROWB,), lambda i: (i,))
    wb = pl.BlockSpec((D, D), lambda i: (0, 0))
    return pl.pallas_call(
        _tca_body,
        grid=grid,
        in_specs=[rb, wb, vb, vb, vb],
        out_specs=[rb, rb, vb, vb],
        out_shape=[
            jax.ShapeDtypeStruct((NP, D), jnp.float32),
            jax.ShapeDtypeStruct((NP, D), jnp.float32),
            jax.ShapeDtypeStruct((NP,), jnp.float32),
            jax.ShapeDtypeStruct((NP,), jnp.float32),
        ],
    )(h, W, degsum2[0], degsum2[1], a)


def _tcb_body(r0_ref, r1_ref, xw_ref, ad_ref, dis2_ref, b_ref, ws_ref, hh_ref, xws_ref, z_ref):
    ad = ad_ref[...]
    hh = jnp.maximum(
        ad[:, None] * (r0_ref[...] + r1_ref[...])
        + xw_ref[...] * dis2_ref[...][:, None] + b_ref[...][None, :],
        0.0,
    )
    hh_ref[...] = hh
    xws = jnp.sum(hh * ws_ref[...][None, :], axis=1)
    xws_ref[...] = xws
    z_ref[...] = xws * ad


def _tcb(ragg2, xw, ad, dis2, b, Ws):
    grid = (_nblocks(),)
    rb = pl.BlockSpec((ROWB, D), lambda i: (i, 0))
    vb = pl.BlockSpec((ROWB,), lambda i: (i,))
    db = pl.BlockSpec((D,), lambda i: (0,))
    return pl.pallas_call(
        _tcb_body,
        grid=grid,
        in_specs=[rb, rb, rb, vb, vb, db, db],
        out_specs=[rb, vb, vb],
        out_shape=[
            jax.ShapeDtypeStruct((NP, D), jnp.float32),
            jax.ShapeDtypeStruct((NP,), jnp.float32),
            jax.ShapeDtypeStruct((NP,), jnp.float32),
        ],
    )(ragg2[0], ragg2[1], xw, ad, dis2, b, Ws[:, 0])


def _tcc_body(k, hh_ref, s0_ref, s1_ref, xws_ref, ad_ref, dis2_ref, a_ref, bs_ref,
              hnext_ref, anext_ref, ro_ref):
    ad = ad_ref[...]
    a = a_ref[...]
    score = ad * (s0_ref[...] + s1_ref[...]) + xws_ref[...] * dis2_ref[...] + bs_ref[0]
    bits = lax.bitcast_convert_type(score, jnp.uint32)
    key = jnp.where(score >= 0, bits | jnp.uint32(0x80000000), ~bits)
    key = jnp.where(a > 0, key, jnp.uint32(0))

    def cnt_ge(t):
        return jnp.sum((key >= t).astype(jnp.int32))

    t = jnp.uint32(0)
    for bit in range(31, -1, -1):
        cand = t | jnp.uint32(1 << bit)
        t = jnp.where(cnt_ge(cand) >= k, cand, t)
    need = k - jnp.sum((key > t).astype(jnp.int32))
    idx = lax.broadcasted_iota(jnp.int32, (NP,), 0)
    iseq = key == t
    u = jnp.int32(0)
    for bit in range(14, -1, -1):
        cand = u + jnp.int32(1 << bit)
        c = jnp.sum((iseq & (idx < cand)).astype(jnp.int32))
        u = jnp.where(c <= need, cand, u)
    kept = (key > t) | (iseq & (idx < u))
    keptf = kept.astype(jnp.float32)
    g = keptf * jnp.tanh(score)
    hn = hh_ref[...] * g[:, None]
    hnext_ref[...] = hn
    anext_ref[...] = keptf
    mx = jnp.max(jnp.where(keptf[:, None] > 0, hn, -jnp.inf), axis=0)
    mn = jnp.sum(hn * keptf[:, None], axis=0) * (1.0 / k)
    ro_ref[0, :D] = mx
    ro_ref[0, D:] = mn


def _tcc(k, hh, sagg2, xws, ad, dis2, a, bs):
    return pl.pallas_call(
        functools.partial(_tcc_body, k),
        out_shape=[
            jax.ShapeDtypeStruct((NP, D), jnp.float32),
            jax.ShapeDtypeStruct((NP,), jnp.float32),
            jax.ShapeDtypeStruct((1, 2 * D), jnp.float32),
        ],
    )(hh, sagg2[0], sagg2[1], xws, ad, dis2, a, bs)


def _tcd_body(s_ref, l1w_ref, l1b_ref, l2w_ref, l2b_ref, l3w_ref, l3b_ref, out_ref):
    s = s_ref[...]
    s = jnp.maximum(jnp.dot(s, l1w_ref[...], preferred_element_type=jnp.float32) + l1b_ref[...][None, :], 0.0)
    s = jnp.maximum(jnp.dot(s, l2w_ref[...], preferred_element_type=jnp.float32) + l2b_ref[...][None, :], 0.0)
    out_ref[...] = jnp.dot(s, l3w_ref[...], preferred_element_type=jnp.float32) + l3b_ref[...][None, :]


def _tcd(s, L1w, L1b, L2w, L2b, L3w, L3b):
    return pl.pallas_call(
        _tcd_body,
        out_shape=jax.ShapeDtypeStruct((1, 10), jnp.float32),
    )(s, L1w, L1b, L2w, L2b, L3w, L3b)


_SC_MESH = plsc.VectorSubcoreMesh(core_axis_name="c", subcore_axis_name="s")


@functools.partial(
    pl.kernel,
    out_type=jax.ShapeDtypeStruct((2, NP), jnp.float32),
    mesh=_SC_MESH,
    scratch_types=[
        pltpu.VMEM((NCHS, CHS), jnp.int32),
        pltpu.VMEM((NCHS, CHS), jnp.int32),
        pltpu.VMEM((FANS * CHS,), jnp.float32),
        pltpu.VMEM_SHARED((NP,), jnp.float32),
        [pltpu.SemaphoreType.DMA] * FANS,
    ],
)
def _sc_seg_scalar(vals_hbm, src_hbm, dst_hbm, zvec_hbm, out_hbm,
                   src_v, dst_v, buf_v, acc_sh, sems):
    cid = lax.axis_index("c")
    sid = lax.axis_index("s")
    wid = sid * NC + cid
    pltpu.sync_copy(zvec_hbm, acc_sh.at[pl.ds(sid * NPS, NPS)])
    pltpu.sync_copy(src_hbm.at[wid], src_v)
    pltpu.sync_copy(dst_hbm.at[wid], dst_v)
    plsc.subcore_barrier()

    def body(jj, carry):
        base = jj * FANS
        cps = [
            pltpu.async_copy(vals_hbm.at[src_v.at[base + b]],
                             buf_v.at[pl.ds(b * CHS, CHS)], sems[b])
            for b in range(FANS)
        ]
        for b in range(FANS):
            cps[b].wait()
            pltpu.sync_copy(buf_v.at[pl.ds(b * CHS, CHS)],
                            acc_sh.at[dst_v.at[base + b]], add=True)
        return carry

    lax.fori_loop(0, NCHS // FANS, body, 0)
    plsc.subcore_barrier()
    pltpu.sync_copy(acc_sh.at[pl.ds(sid * NPS, NPS)],
                    out_hbm.at[cid, pl.ds(sid * NPS, NPS)])


@functools.partial(
    pl.kernel,
    out_type=jax.ShapeDtypeStruct((2, NP, D), jnp.float32),
    mesh=_SC_MESH,
    scratch_types=[
        pltpu.VMEM((NCHR, CHR), jnp.int32),
        pltpu.VMEM((NCHR, CHR), jnp.int32),
        pltpu.VMEM((CHR, D), jnp.float32),
        pltpu.VMEM_SHARED((NP, D), jnp.float32),
        pltpu.SemaphoreType.DMA,
    ],
)
def _sc_seg_rows(y_hbm, src_hbm, dst_hbm, zrows_hbm, out_hbm,
                 src_v, dst_v, rows_v, acc_sh, sem):
    cid = lax.axis_index("c")
    sid = lax.axis_index("s")
    wid = sid * NC + cid
    pltpu.sync_copy(zrows_hbm, acc_sh.at[pl.ds(sid * NPS, NPS)])
    pltpu.sync_copy(src_hbm.at[wid], src_v)
    pltpu.sync_copy(dst_hbm.at[wid], dst_v)
    plsc.subcore_barrier()

    def body(j, carry):
        pltpu.async_copy(y_hbm.at[src_v.at[j]], rows_v, sem).wait()
        pltpu.sync_copy(rows_v, acc_sh.at[dst_v.at[j]], add=True)
        return carry

    lax.fori_loop(0, NCHR, body, 0)
    plsc.subcore_barrier()
    pltpu.sync_copy(acc_sh.at[pl.ds(sid * NPS, NPS)],
                    out_hbm.at[cid, pl.ds(sid * NPS, NPS)])


def kernel(x, edge_index, batch, W1, b1, Ws1, bs1, W2, b2, Ws2, bs2, W3, b3,
           Ws3, bs3, L1w, L1b, L2w, L2b, L3w, L3b):
    npad = E2 - E
    piota = jnp.arange(npad, dtype=edge_index.dtype)
    pad_src = N + piota % (NP - N)
    pad_dst = piota % NP
    srcp = jnp.concatenate([edge_index[0], pad_src])
    dstp = jnp.concatenate([edge_index[1], pad_dst])
    src3 = jnp.reshape(srcp, (NW, NCHS, CHS))
    dst3 = jnp.reshape(dstp, (NW, NCHS, CHS))
    src2r = jnp.reshape(srcp, (NW, NCHR, CHR))
    dst2r = jnp.reshape(dstp, (NW, NCHR, CHR))
    zvec = jnp.zeros((NPS,), jnp.float32)
    zrows = jnp.zeros((NPS, D), jnp.float32)
    h = jnp.pad(x, ((0, NP - N), (0, 0)))
    a = jnp.pad(jnp.ones((N,), jnp.float32), (0, NP - N))

    ks = []
    kk = N
    for _ in range(3):
        kk = int(math.ceil(0.8 * kk))
        ks.append(kk)

    params = [(W1, b1, Ws1, bs1), (W2, b2, Ws2, bs2), (W3, b3, Ws3, bs3)]
    readouts = []
    for r in range(3):
        W, b, Wsc, bsc = params[r]
        k = ks[r]
        degsum2 = _sc_seg_scalar(a, src3, dst3, zvec)
        xw, y, ad, dis2 = _tca(h, W, degsum2, a)
        ragg2 = _sc_seg_rows(y, src2r, dst2r, zrows)
        hh, xws, z = _tcb(ragg2, xw, ad, dis2, b, Wsc)
        sagg2 = _sc_seg_scalar(z, src3, dst3, zvec)
        h, a, ro = _tcc(k, hh, sagg2, xws, ad, dis2, a, bsc)
        readouts.append(ro)

    s = readouts[0] + readouts[1] + readouts[2]
    return _tcd(s, L1w, L1b, L2w, L2b, L3w, L3b)


# speedup vs baseline: 2.2907x; 1.0724x over previous
import functools
import math

import jax
import jax.numpy as jnp
from jax import lax
from jax.experimental import pallas as pl
from jax.experimental.pallas import tpu as pltpu
from jax.experimental.pallas import tpu_sc as plsc

N = 10000
E = 320000
NP = 10240
D = 128
ROWB = 256

NC = 2
NS = 16
NW = NC * NS
EPW = 10240
E2 = NW * EPW
CHS = 128
NCHS = EPW // CHS
FANS = 8
CHR = 128
NCHR = EPW // CHR
FANR = 2
NPS = NP // NS


def _nblocks():
    return NP // ROWB


def _tca_body(h_ref, w_ref, d0_ref, d1_ref, a_ref, xw_ref, y2_ref, ad_ref, dis2_ref):
    a = a_ref[...]
    deg = a * (d0_ref[...] + d1_ref[...]) + 1.0
    dis2 = 1.0 / deg
    dis = jnp.sqrt(dis2)
    ad = a * dis
    xw = jnp.dot(h_ref[...], w_ref[...], preferred_element_type=jnp.float32)
    xw_ref[...] = xw
    y2_ref[...] = xw * ad[:, None]
    ad_ref[...] = ad
    dis2_ref[...] = dis2


def _tca(h, W, degsum2, a):
    grid = (_nblocks(),)
    rb = pl.BlockSpec((ROWB, D), lambda i: (i, 0))
    vb = pl.BlockSpec((ROWB,), lambda i: (i,))
    wb = pl.BlockSpec((D, D), lambda i: (0, 0))
    return pl.pallas_call(
        _tca_body,
        grid=grid,
        in_specs=[rb, wb, vb, vb, vb],
        out_specs=[rb, rb, vb, vb],
        out_shape=[
            jax.ShapeDtypeStruct((NP, D), jnp.float32),
            jax.ShapeDtypeStruct((NP, D), jnp.float32),
            jax.ShapeDtypeStruct((NP,), jnp.float32),
            jax.ShapeDtypeStruct((NP,), jnp.float32),
        ],
    )(h, W, degsum2[0], degsum2[1], a)


def _tcb_body(r0_ref, r1_ref, xw_ref, ad_ref, dis2_ref, b_ref, ws_ref, hh_ref, xws_ref, z_ref):
    ad = ad_ref[...]
    hh = jnp.maximum(
        ad[:, None] * (r0_ref[...] + r1_ref[...])
        + xw_ref[...] * dis2_ref[...][:, None] + b_ref[...][None, :],
        0.0,
    )
    hh_ref[...] = hh
    xws = jnp.sum(hh * ws_ref[...][None, :], axis=1)
    xws_ref[...] = xws
    z_ref[...] = xws * ad


def _tcb(ragg2, xw, ad, dis2, b, Ws):
    grid = (_nblocks(),)
    rb = pl.BlockSpec((ROWB, D), lambda i: (i, 0))
    vb = pl.BlockSpec((ROWB,), lambda i: (i,))
    db = pl.BlockSpec((D,), lambda i: (0,))
    return pl.pallas_call(
        _tcb_body,
        grid=grid,
        in_specs=[rb, rb, rb, vb, vb, db, db],
        out_specs=[rb, vb, vb],
        out_shape=[
            jax.ShapeDtypeStruct((NP, D), jnp.float32),
            jax.ShapeDtypeStruct((NP,), jnp.float32),
            jax.ShapeDtypeStruct((NP,), jnp.float32),
        ],
    )(ragg2[0], ragg2[1], xw, ad, dis2, b, Ws[:, 0])


def _tcc_body(k, hh_ref, s0_ref, s1_ref, xws_ref, ad_ref, dis2_ref, a_ref, bs_ref,
              hnext_ref, anext_ref, ro_ref):
    ad = ad_ref[...]
    a = a_ref[...]
    score = ad * (s0_ref[...] + s1_ref[...]) + xws_ref[...] * dis2_ref[...] + bs_ref[0]
    bits = lax.bitcast_convert_type(score, jnp.uint32)
    key = jnp.where(score >= 0, bits | jnp.uint32(0x80000000), ~bits)
    key = jnp.where(a > 0, key, jnp.uint32(0))

    def cnt_ge(t):
        return jnp.sum((key >= t).astype(jnp.int32))

    t = jnp.uint32(0)
    for bit in range(31, -1, -1):
        cand = t | jnp.uint32(1 << bit)
        t = jnp.where(cnt_ge(cand) >= k, cand, t)
    need = k - jnp.sum((key > t).astype(jnp.int32))
    idx = lax.broadcasted_iota(jnp.int32, (NP,), 0)
    iseq = key == t
    u = jnp.int32(0)
    for bit in range(14, -1, -1):
        cand = u + jnp.int32(1 << bit)
        c = jnp.sum((iseq & (idx < cand)).astype(jnp.int32))
        u = jnp.where(c <= need, cand, u)
    kept = (key > t) | (iseq & (idx < u))
    keptf = kept.astype(jnp.float32)
    g = keptf * jnp.tanh(score)
    hn = hh_ref[...] * g[:, None]
    hnext_ref[...] = hn
    anext_ref[...] = keptf
    mx = jnp.max(jnp.where(keptf[:, None] > 0, hn, -jnp.inf), axis=0)
    mn = jnp.sum(hn * keptf[:, None], axis=0) * (1.0 / k)
    ro_ref[0, :D] = mx
    ro_ref[0, D:] = mn


def _tcc(k, hh, sagg2, xws, ad, dis2, a, bs):
    return pl.pallas_call(
        functools.partial(_tcc_body, k),
        out_shape=[
            jax.ShapeDtypeStruct((NP, D), jnp.float32),
            jax.ShapeDtypeStruct((NP,), jnp.float32),
            jax.ShapeDtypeStruct((1, 2 * D), jnp.float32),
        ],
    )(hh, sagg2[0], sagg2[1], xws, ad, dis2, a, bs)


def _tcd_body(s_ref, l1w_ref, l1b_ref, l2w_ref, l2b_ref, l3w_ref, l3b_ref, out_ref):
    s = s_ref[...]
    s = jnp.maximum(jnp.dot(s, l1w_ref[...], preferred_element_type=jnp.float32) + l1b_ref[...][None, :], 0.0)
    s = jnp.maximum(jnp.dot(s, l2w_ref[...], preferred_element_type=jnp.float32) + l2b_ref[...][None, :], 0.0)
    out_ref[...] = jnp.dot(s, l3w_ref[...], preferred_element_type=jnp.float32) + l3b_ref[...][None, :]


def _tcd(s, L1w, L1b, L2w, L2b, L3w, L3b):
    return pl.pallas_call(
        _tcd_body,
        out_shape=jax.ShapeDtypeStruct((1, 10), jnp.float32),
    )(s, L1w, L1b, L2w, L2b, L3w, L3b)


_SC_MESH = plsc.VectorSubcoreMesh(core_axis_name="c", subcore_axis_name="s")


@functools.partial(
    pl.kernel,
    out_type=jax.ShapeDtypeStruct((2, NP), jnp.float32),
    mesh=_SC_MESH,
    scratch_types=[
        pltpu.VMEM((NCHS, CHS), jnp.int32),
        pltpu.VMEM((NCHS, CHS), jnp.int32),
        pltpu.VMEM((FANS * CHS,), jnp.float32),
        pltpu.VMEM_SHARED((NP,), jnp.float32),
        [pltpu.SemaphoreType.DMA] * FANS,
    ],
)
def _sc_seg_scalar(vals_hbm, src_hbm, dst_hbm, zvec_hbm, out_hbm,
                   src_v, dst_v, buf_v, acc_sh, sems):
    cid = lax.axis_index("c")
    sid = lax.axis_index("s")
    wid = sid * NC + cid
    pltpu.sync_copy(zvec_hbm, acc_sh.at[pl.ds(sid * NPS, NPS)])
    pltpu.sync_copy(src_hbm.at[wid], src_v)
    pltpu.sync_copy(dst_hbm.at[wid], dst_v)
    plsc.subcore_barrier()

    def body(jj, carry):
        base = jj * FANS
        cps = [
            pltpu.async_copy(vals_hbm.at[src_v.at[base + b]],
                             buf_v.at[pl.ds(b * CHS, CHS)], sems[b])
            for b in range(FANS)
        ]
        for b in range(FANS):
            cps[b].wait()
            pltpu.sync_copy(buf_v.at[pl.ds(b * CHS, CHS)],
                            acc_sh.at[dst_v.at[base + b]], add=True)
        return carry

    lax.fori_loop(0, NCHS // FANS, body, 0)
    plsc.subcore_barrier()
    pltpu.sync_copy(acc_sh.at[pl.ds(sid * NPS, NPS)],
                    out_hbm.at[cid, pl.ds(sid * NPS, NPS)])


@functools.partial(
    pl.kernel,
    out_type=jax.ShapeDtypeStruct((2, NP, D), jnp.float32),
    mesh=_SC_MESH,
    scratch_types=[
        pltpu.VMEM((FANR, CHR), jnp.int32),
        pltpu.VMEM((FANR, CHR), jnp.int32),
        pltpu.VMEM((FANR * CHR, D), jnp.float32),
        pltpu.VMEM_SHARED((NP, D), jnp.float32),
        [pltpu.SemaphoreType.DMA] * FANR,
    ],
)
def _sc_seg_rows(y_hbm, src_hbm, dst_hbm, zrows_hbm, out_hbm,
                 src_v, dst_v, rows_v, acc_sh, sems):
    cid = lax.axis_index("c")
    sid = lax.axis_index("s")
    wid = sid * NC + cid
    pltpu.sync_copy(zrows_hbm, acc_sh.at[pl.ds(sid * NPS, NPS)])
    plsc.subcore_barrier()

    def body(jj, carry):
        base = jj * FANR
        cps = []
        for b in range(FANR):
            pltpu.sync_copy(src_hbm.at[wid, base + b], src_v.at[b])
            pltpu.sync_copy(dst_hbm.at[wid, base + b], dst_v.at[b])
            cps.append(pltpu.async_copy(y_hbm.at[src_v.at[b]],
                                        rows_v.at[pl.ds(b * CHR, CHR)],
                                        sems[b]))
        for b in range(FANR):
            cps[b].wait()
            pltpu.sync_copy(rows_v.at[pl.ds(b * CHR, CHR)],
                            acc_sh.at[dst_v.at[b]], add=True)
        return carry

    lax.fori_loop(0, NCHR // FANR, body, 0)
    plsc.subcore_barrier()
    pltpu.sync_copy(acc_sh.at[pl.ds(sid * NPS, NPS)],
                    out_hbm.at[cid, pl.ds(sid * NPS, NPS)])


def kernel(x, edge_index, batch, W1, b1, Ws1, bs1, W2, b2, Ws2, bs2, W3, b3,
           Ws3, bs3, L1w, L1b, L2w, L2b, L3w, L3b):
    npad = E2 - E
    piota = jnp.arange(npad, dtype=edge_index.dtype)
    pad_src = N + piota % (NP - N)
    pad_dst = piota % NP
    srcp = jnp.concatenate([edge_index[0], pad_src])
    dstp = jnp.concatenate([edge_index[1], pad_dst])
    src3 = jnp.reshape(srcp, (NW, NCHS, CHS))
    dst3 = jnp.reshape(dstp, (NW, NCHS, CHS))
    src2r = jnp.reshape(srcp, (NW, NCHR, CHR))
    dst2r = jnp.reshape(dstp, (NW, NCHR, CHR))
    zvec = jnp.zeros((NPS,), jnp.float32)
    zrows = jnp.zeros((NPS, D), jnp.float32)
    h = jnp.pad(x, ((0, NP - N), (0, 0)))
    a = jnp.pad(jnp.ones((N,), jnp.float32), (0, NP - N))

    ks = []
    kk = N
    for _ in range(3):
        kk = int(math.ceil(0.8 * kk))
        ks.append(kk)

    params = [(W1, b1, Ws1, bs1), (W2, b2, Ws2, bs2), (W3, b3, Ws3, bs3)]
    readouts = []
    for r in range(3):
        W, b, Wsc, bsc = params[r]
        k = ks[r]
        degsum2 = _sc_seg_scalar(a, src3, dst3, zvec)
        xw, y, ad, dis2 = _tca(h, W, degsum2, a)
        ragg2 = _sc_seg_rows(y, src2r, dst2r, zrows)
        hh, xws, z = _tcb(ragg2, xw, ad, dis2, b, Wsc)
        sagg2 = _sc_seg_scalar(z, src3, dst3, zvec)
        h, a, ro = _tcc(k, hh, sagg2, xws, ad, dis2, a, bsc)
        readouts.append(ro)

    s = readouts[0] + readouts[1] + readouts[2]
    return _tcd(s, L1w, L1b, L2w, L2b, L3w, L3b)


# speedup vs baseline: 2.2910x; 1.0001x over previous
import functools
import math

import jax
import jax.numpy as jnp
from jax import lax
from jax.experimental import pallas as pl
from jax.experimental.pallas import tpu as pltpu
from jax.experimental.pallas import tpu_sc as plsc

N = 10000
E = 320000
NP = 10240
D = 128
ROWB = 256

NC = 2
NS = 16
NW = NC * NS
EPW = 10240
E2 = NW * EPW
CHS = 128
NCHS = EPW // CHS
FANS = 16
CHR = 128
NCHR = EPW // CHR
FANR = 2
NPS = NP // NS


def _nblocks():
    return NP // ROWB


def _tca_body(h_ref, w_ref, d0_ref, d1_ref, a_ref, xw_ref, y2_ref, ad_ref, dis2_ref):
    a = a_ref[...]
    deg = a * (d0_ref[...] + d1_ref[...]) + 1.0
    dis2 = 1.0 / deg
    dis = jnp.sqrt(dis2)
    ad = a * dis
    xw = jnp.dot(h_ref[...], w_ref[...], preferred_element_type=jnp.float32)
    xw_ref[...] = xw
    y2_ref[...] = xw * ad[:, None]
    ad_ref[...] = ad
    dis2_ref[...] = dis2


def _tca(h, W, degsum2, a):
    grid = (_nblocks(),)
    rb = pl.BlockSpec((ROWB, D), lambda i: (i, 0))
    vb = pl.BlockSpec((ROWB,), lambda i: (i,))
    wb = pl.BlockSpec((D, D), lambda i: (0, 0))
    return pl.pallas_call(
        _tca_body,
        grid=grid,
        in_specs=[rb, wb, vb, vb, vb],
        out_specs=[rb, rb, vb, vb],
        out_shape=[
            jax.ShapeDtypeStruct((NP, D), jnp.float32),
            jax.ShapeDtypeStruct((NP, D), jnp.float32),
            jax.ShapeDtypeStruct((NP,), jnp.float32),
            jax.ShapeDtypeStruct((NP,), jnp.float32),
        ],
    )(h, W, degsum2[0], degsum2[1], a)


def _tcb_body(r0_ref, r1_ref, xw_ref, ad_ref, dis2_ref, b_ref, ws_ref, hh_ref, xws_ref, z_ref):
    ad = ad_ref[...]
    hh = jnp.maximum(
        ad[:, None] * (r0_ref[...] + r1_ref[...])
        + xw_ref[...] * dis2_ref[...][:, None] + b_ref[...][None, :],
        0.0,
    )
    hh_ref[...] = hh
    xws = jnp.sum(hh * ws_ref[...][None, :], axis=1)
    xws_ref[...] = xws
    z_ref[...] = xws * ad


def _tcb(ragg2, xw, ad, dis2, b, Ws):
    grid = (_nblocks(),)
    rb = pl.BlockSpec((ROWB, D), lambda i: (i, 0))
    vb = pl.BlockSpec((ROWB,), lambda i: (i,))
    db = pl.BlockSpec((D,), lambda i: (0,))
    return pl.pallas_call(
        _tcb_body,
        grid=grid,
        in_specs=[rb, rb, rb, vb, vb, db, db],
        out_specs=[rb, vb, vb],
        out_shape=[
            jax.ShapeDtypeStruct((NP, D), jnp.float32),
            jax.ShapeDtypeStruct((NP,), jnp.float32),
            jax.ShapeDtypeStruct((NP,), jnp.float32),
        ],
    )(ragg2[0], ragg2[1], xw, ad, dis2, b, Ws[:, 0])


def _tcc_body(k, hh_ref, s0_ref, s1_ref, xws_ref, ad_ref, dis2_ref, a_ref, bs_ref,
              hnext_ref, anext_ref, ro_ref):
    ad = ad_ref[...]
    a = a_ref[...]
    score = ad * (s0_ref[...] + s1_ref[...]) + xws_ref[...] * dis2_ref[...] + bs_ref[0]
    bits = lax.bitcast_convert_type(score, jnp.uint32)
    key = jnp.where(score >= 0, bits | jnp.uint32(0x80000000), ~bits)
    key = jnp.where(a > 0, key, jnp.uint32(0))

    def cnt_ge(t):
        return jnp.sum((key >= t).astype(jnp.int32))

    t = jnp.uint32(0)
    for bit in range(31, -1, -1):
        cand = t | jnp.uint32(1 << bit)
        t = jnp.where(cnt_ge(cand) >= k, cand, t)
    need = k - jnp.sum((key > t).astype(jnp.int32))
    idx = lax.broadcasted_iota(jnp.int32, (NP,), 0)
    iseq = key == t
    u = jnp.int32(0)
    for bit in range(14, -1, -1):
        cand = u + jnp.int32(1 << bit)
        c = jnp.sum((iseq & (idx < cand)).astype(jnp.int32))
        u = jnp.where(c <= need, cand, u)
    kept = (key > t) | (iseq & (idx < u))
    keptf = kept.astype(jnp.float32)
    g = keptf * jnp.tanh(score)
    hn = hh_ref[...] * g[:, None]
    hnext_ref[...] = hn
    anext_ref[...] = keptf
    mx = jnp.max(jnp.where(keptf[:, None] > 0, hn, -jnp.inf), axis=0)
    mn = jnp.sum(hn * keptf[:, None], axis=0) * (1.0 / k)
    ro_ref[0, :D] = mx
    ro_ref[0, D:] = mn


def _tcc(k, hh, sagg2, xws, ad, dis2, a, bs):
    return pl.pallas_call(
        functools.partial(_tcc_body, k),
        out_shape=[
            jax.ShapeDtypeStruct((NP, D), jnp.float32),
            jax.ShapeDtypeStruct((NP,), jnp.float32),
            jax.ShapeDtypeStruct((1, 2 * D), jnp.float32),
        ],
    )(hh, sagg2[0], sagg2[1], xws, ad, dis2, a, bs)


def _tcd_body(s_ref, l1w_ref, l1b_ref, l2w_ref, l2b_ref, l3w_ref, l3b_ref, out_ref):
    s = s_ref[...]
    s = jnp.maximum(jnp.dot(s, l1w_ref[...], preferred_element_type=jnp.float32) + l1b_ref[...][None, :], 0.0)
    s = jnp.maximum(jnp.dot(s, l2w_ref[...], preferred_element_type=jnp.float32) + l2b_ref[...][None, :], 0.0)
    out_ref[...] = jnp.dot(s, l3w_ref[...], preferred_element_type=jnp.float32) + l3b_ref[...][None, :]


def _tcd(s, L1w, L1b, L2w, L2b, L3w, L3b):
    return pl.pallas_call(
        _tcd_body,
        out_shape=jax.ShapeDtypeStruct((1, 10), jnp.float32),
    )(s, L1w, L1b, L2w, L2b, L3w, L3b)


_SC_MESH = plsc.VectorSubcoreMesh(core_axis_name="c", subcore_axis_name="s")


@functools.partial(
    pl.kernel,
    out_type=jax.ShapeDtypeStruct((2, NP), jnp.float32),
    mesh=_SC_MESH,
    scratch_types=[
        pltpu.VMEM((NCHS, CHS), jnp.int32),
        pltpu.VMEM((NCHS, CHS), jnp.int32),
        pltpu.VMEM((FANS * CHS,), jnp.float32),
        pltpu.VMEM_SHARED((NP,), jnp.float32),
        [pltpu.SemaphoreType.DMA] * FANS,
    ],
)
def _sc_seg_scalar(vals_hbm, src_hbm, dst_hbm, zvec_hbm, out_hbm,
                   src_v, dst_v, buf_v, acc_sh, sems):
    cid = lax.axis_index("c")
    sid = lax.axis_index("s")
    wid = sid * NC + cid
    pltpu.sync_copy(zvec_hbm, acc_sh.at[pl.ds(sid * NPS, NPS)])
    pltpu.sync_copy(src_hbm.at[wid], src_v)
    pltpu.sync_copy(dst_hbm.at[wid], dst_v)
    plsc.subcore_barrier()

    def body(jj, carry):
        base = jj * FANS
        cps = [
            pltpu.async_copy(vals_hbm.at[src_v.at[base + b]],
                             buf_v.at[pl.ds(b * CHS, CHS)], sems[b])
            for b in range(FANS)
        ]
        for b in range(FANS):
            cps[b].wait()
            pltpu.sync_copy(buf_v.at[pl.ds(b * CHS, CHS)],
                            acc_sh.at[dst_v.at[base + b]], add=True)
        return carry

    lax.fori_loop(0, NCHS // FANS, body, 0)
    plsc.subcore_barrier()
    pltpu.sync_copy(acc_sh.at[pl.ds(sid * NPS, NPS)],
                    out_hbm.at[cid, pl.ds(sid * NPS, NPS)])


@functools.partial(
    pl.kernel,
    out_type=jax.ShapeDtypeStruct((2, NP, D), jnp.float32),
    mesh=_SC_MESH,
    scratch_types=[
        pltpu.VMEM((FANR, CHR), jnp.int32),
        pltpu.VMEM((FANR, CHR), jnp.int32),
        pltpu.VMEM((FANR * CHR, D), jnp.float32),
        pltpu.VMEM_SHARED((NP, D), jnp.float32),
        [pltpu.SemaphoreType.DMA] * FANR,
    ],
)
def _sc_seg_rows(y_hbm, src_hbm, dst_hbm, zrows_hbm, out_hbm,
                 src_v, dst_v, rows_v, acc_sh, sems):
    cid = lax.axis_index("c")
    sid = lax.axis_index("s")
    wid = sid * NC + cid
    pltpu.sync_copy(zrows_hbm, acc_sh.at[pl.ds(sid * NPS, NPS)])
    plsc.subcore_barrier()

    def body(jj, carry):
        base = jj * FANR
        cps = []
        for b in range(FANR):
            pltpu.sync_copy(src_hbm.at[wid, base + b], src_v.at[b])
            pltpu.sync_copy(dst_hbm.at[wid, base + b], dst_v.at[b])
            cps.append(pltpu.async_copy(y_hbm.at[src_v.at[b]],
                                        rows_v.at[pl.ds(b * CHR, CHR)],
                                        sems[b]))
        for b in range(FANR):
            cps[b].wait()
            pltpu.sync_copy(rows_v.at[pl.ds(b * CHR, CHR)],
                            acc_sh.at[dst_v.at[b]], add=True)
        return carry

    lax.fori_loop(0, NCHR // FANR, body, 0)
    plsc.subcore_barrier()
    pltpu.sync_copy(acc_sh.at[pl.ds(sid * NPS, NPS)],
                    out_hbm.at[cid, pl.ds(sid * NPS, NPS)])


def kernel(x, edge_index, batch, W1, b1, Ws1, bs1, W2, b2, Ws2, bs2, W3, b3,
           Ws3, bs3, L1w, L1b, L2w, L2b, L3w, L3b):
    npad = E2 - E
    piota = jnp.arange(npad, dtype=edge_index.dtype)
    pad_src = N + piota % (NP - N)
    pad_dst = piota % NP
    srcp = jnp.concatenate([edge_index[0], pad_src])
    dstp = jnp.concatenate([edge_index[1], pad_dst])
    src3 = jnp.reshape(srcp, (NW, NCHS, CHS))
    dst3 = jnp.reshape(dstp, (NW, NCHS, CHS))
    src2r = jnp.reshape(srcp, (NW, NCHR, CHR))
    dst2r = jnp.reshape(dstp, (NW, NCHR, CHR))
    zvec = jnp.zeros((NPS,), jnp.float32)
    zrows = jnp.zeros((NPS, D), jnp.float32)
    h = jnp.pad(x, ((0, NP - N), (0, 0)))
    a = jnp.pad(jnp.ones((N,), jnp.float32), (0, NP - N))

    ks = []
    kk = N
    for _ in range(3):
        kk = int(math.ceil(0.8 * kk))
        ks.append(kk)

    params = [(W1, b1, Ws1, bs1), (W2, b2, Ws2, bs2), (W3, b3, Ws3, bs3)]
    readouts = []
    for r in range(3):
        W, b, Wsc, bsc = params[r]
        k = ks[r]
        degsum2 = _sc_seg_scalar(a, src3, dst3, zvec)
        xw, y, ad, dis2 = _tca(h, W, degsum2, a)
        ragg2 = _sc_seg_rows(y, src2r, dst2r, zrows)
        hh, xws, z = _tcb(ragg2, xw, ad, dis2, b, Wsc)
        sagg2 = _sc_seg_scalar(z, src3, dst3, zvec)
        h, a, ro = _tcc(k, hh, sagg2, xws, ad, dis2, a, bsc)
        readouts.append(ro)

    s = readouts[0] + readouts[1] + readouts[2]
    return _tcd(s, L1w, L1b, L2w, L2b, L3w, L3b)


# speedup vs baseline: 2.3030x; 1.0052x over previous
import functools
import math

import jax
import jax.numpy as jnp
from jax import lax
from jax.experimental import pallas as pl
from jax.experimental.pallas import tpu as pltpu
from jax.experimental.pallas import tpu_sc as plsc

N = 10000
E = 320000
NP = 10240
D = 128
ROWB = 256

NC = 2
NS = 16
NW = NC * NS
EPW = 10240
E2 = NW * EPW
CHS = 128
NCHS = EPW // CHS
FANS = 16
CHR = 128
NCHR = EPW // CHR
FANR = 2
NPS = NP // NS


def _nblocks():
    return NP // ROWB


def _tcx_body(h_ref, w_ref, xw_ref):
    xw_ref[...] = jnp.dot(h_ref[...], w_ref[...], preferred_element_type=jnp.float32)


def _tcx(h, W):
    grid = (_nblocks(),)
    rb = pl.BlockSpec((ROWB, D), lambda i: (i, 0))
    wb = pl.BlockSpec((D, D), lambda i: (0, 0))
    return pl.pallas_call(
        _tcx_body,
        grid=grid,
        in_specs=[rb, wb],
        out_specs=rb,
        out_shape=jax.ShapeDtypeStruct((NP, D), jnp.float32),
    )(h, W)


def _tca_body(xw_ref, d0_ref, d1_ref, a_ref, y2_ref, ad_ref, dis2_ref):
    a = a_ref[...]
    deg = a * (d0_ref[...] + d1_ref[...]) + 1.0
    dis2 = 1.0 / deg
    dis = jnp.sqrt(dis2)
    ad = a * dis
    y2_ref[...] = xw_ref[...] * ad[:, None]
    ad_ref[...] = ad
    dis2_ref[...] = dis2


def _tca(xw, degsum2, a):
    grid = (_nblocks(),)
    rb = pl.BlockSpec((ROWB, D), lambda i: (i, 0))
    vb = pl.BlockSpec((ROWB,), lambda i: (i,))
    return pl.pallas_call(
        _tca_body,
        grid=grid,
        in_specs=[rb, vb, vb, vb],
        out_specs=[rb, vb, vb],
        out_shape=[
            jax.ShapeDtypeStruct((NP, D), jnp.float32),
            jax.ShapeDtypeStruct((NP,), jnp.float32),
            jax.ShapeDtypeStruct((NP,), jnp.float32),
        ],
    )(xw, degsum2[0], degsum2[1], a)


def _tcb_body(r0_ref, r1_ref, xw_ref, ad_ref, dis2_ref, b_ref, ws_ref, hh_ref, xws_ref, z_ref):
    ad = ad_ref[...]
    hh = jnp.maximum(
        ad[:, None] * (r0_ref[...] + r1_ref[...])
        + xw_ref[...] * dis2_ref[...][:, None] + b_ref[...][None, :],
        0.0,
    )
    hh_ref[...] = hh
    xws = jnp.sum(hh * ws_ref[...][None, :], axis=1)
    xws_ref[...] = xws
    z_ref[...] = xws * ad


def _tcb(ragg2, xw, ad, dis2, b, Ws):
    grid = (_nblocks(),)
    rb = pl.BlockSpec((ROWB, D), lambda i: (i, 0))
    vb = pl.BlockSpec((ROWB,), lambda i: (i,))
    db = pl.BlockSpec((D,), lambda i: (0,))
    return pl.pallas_call(
        _tcb_body,
        grid=grid,
        in_specs=[rb, rb, rb, vb, vb, db, db],
        out_specs=[rb, vb, vb],
        out_shape=[
            jax.ShapeDtypeStruct((NP, D), jnp.float32),
            jax.ShapeDtypeStruct((NP,), jnp.float32),
            jax.ShapeDtypeStruct((NP,), jnp.float32),
        ],
    )(ragg2[0], ragg2[1], xw, ad, dis2, b, Ws[:, 0])


def _tcc_body(k, hh_ref, s0_ref, s1_ref, xws_ref, ad_ref, dis2_ref, a_ref, bs_ref,
              hnext_ref, anext_ref, ro_ref):
    ad = ad_ref[...]
    a = a_ref[...]
    score = ad * (s0_ref[...] + s1_ref[...]) + xws_ref[...] * dis2_ref[...] + bs_ref[0]
    bits = lax.bitcast_convert_type(score, jnp.uint32)
    key = jnp.where(score >= 0, bits | jnp.uint32(0x80000000), ~bits)
    key = jnp.where(a > 0, key, jnp.uint32(0))

    def cnt_ge(t):
        return jnp.sum((key >= t).astype(jnp.int32))

    t = jnp.uint32(0)
    for bit in range(31, -1, -1):
        cand = t | jnp.uint32(1 << bit)
        t = jnp.where(cnt_ge(cand) >= k, cand, t)
    need = k - jnp.sum((key > t).astype(jnp.int32))
    idx = lax.broadcasted_iota(jnp.int32, (NP,), 0)
    iseq = key == t
    u = jnp.int32(0)
    for bit in range(14, -1, -1):
        cand = u + jnp.int32(1 << bit)
        c = jnp.sum((iseq & (idx < cand)).astype(jnp.int32))
        u = jnp.where(c <= need, cand, u)
    kept = (key > t) | (iseq & (idx < u))
    keptf = kept.astype(jnp.float32)
    g = keptf * jnp.tanh(score)
    hn = hh_ref[...] * g[:, None]
    hnext_ref[...] = hn
    anext_ref[...] = keptf
    mx = jnp.max(jnp.where(keptf[:, None] > 0, hn, -jnp.inf), axis=0)
    mn = jnp.sum(hn * keptf[:, None], axis=0) * (1.0 / k)
    ro_ref[0, :D] = mx
    ro_ref[0, D:] = mn


def _tcc(k, hh, sagg2, xws, ad, dis2, a, bs):
    return pl.pallas_call(
        functools.partial(_tcc_body, k),
        out_shape=[
            jax.ShapeDtypeStruct((NP, D), jnp.float32),
            jax.ShapeDtypeStruct((NP,), jnp.float32),
            jax.ShapeDtypeStruct((1, 2 * D), jnp.float32),
        ],
    )(hh, sagg2[0], sagg2[1], xws, ad, dis2, a, bs)


def _tcc3_body(k, hh_ref, s0_ref, s1_ref, xws_ref, ad_ref, dis2_ref, a_ref, bs_ref,
               ro1_ref, ro2_ref, l1w_ref, l1b_ref, l2w_ref, l2b_ref, l3w_ref,
               l3b_ref, out_ref):
    ad = ad_ref[...]
    a = a_ref[...]
    score = ad * (s0_ref[...] + s1_ref[...]) + xws_ref[...] * dis2_ref[...] + bs_ref[0]
    bits = lax.bitcast_convert_type(score, jnp.uint32)
    key = jnp.where(score >= 0, bits | jnp.uint32(0x80000000), ~bits)
    key = jnp.where(a > 0, key, jnp.uint32(0))

    t = jnp.uint32(0)
    for bit in range(31, -1, -1):
        cand = t | jnp.uint32(1 << bit)
        t = jnp.where(jnp.sum((key >= cand).astype(jnp.int32)) >= k, cand, t)
    need = k - jnp.sum((key > t).astype(jnp.int32))
    idx = lax.broadcasted_iota(jnp.int32, (NP,), 0)
    iseq = key == t
    u = jnp.int32(0)
    for bit in range(14, -1, -1):
        cand = u + jnp.int32(1 << bit)
        c = jnp.sum((iseq & (idx < cand)).astype(jnp.int32))
        u = jnp.where(c <= need, cand, u)
    kept = (key > t) | (iseq & (idx < u))
    keptf = kept.astype(jnp.float32)
    g = keptf * jnp.tanh(score)
    hn = hh_ref[...] * g[:, None]
    mx = jnp.max(jnp.where(keptf[:, None] > 0, hn, -jnp.inf), axis=0)
    mn = jnp.sum(hn * keptf[:, None], axis=0) * (1.0 / k)
    s = ro1_ref[...] + ro2_ref[...]
    s = s + jnp.concatenate([mx[None, :], mn[None, :]], axis=1)
    s = jnp.maximum(jnp.dot(s, l1w_ref[...], preferred_element_type=jnp.float32) + l1b_ref[...][None, :], 0.0)
    s = jnp.maximum(jnp.dot(s, l2w_ref[...], preferred_element_type=jnp.float32) + l2b_ref[...][None, :], 0.0)
    out_ref[...] = jnp.dot(s, l3w_ref[...], preferred_element_type=jnp.float32) + l3b_ref[...][None, :]


def _tcc3(k, hh, sagg2, xws, ad, dis2, a, bs, ro1, ro2, mlp):
    return pl.pallas_call(
        functools.partial(_tcc3_body, k),
        out_shape=jax.ShapeDtypeStruct((1, 10), jnp.float32),
    )(hh, sagg2[0], sagg2[1], xws, ad, dis2, a, bs, ro1, ro2, *mlp)


def _tcd_body(s_ref, l1w_ref, l1b_ref, l2w_ref, l2b_ref, l3w_ref, l3b_ref, out_ref):
    s = s_ref[...]
    s = jnp.maximum(jnp.dot(s, l1w_ref[...], preferred_element_type=jnp.float32) + l1b_ref[...][None, :], 0.0)
    s = jnp.maximum(jnp.dot(s, l2w_ref[...], preferred_element_type=jnp.float32) + l2b_ref[...][None, :], 0.0)
    out_ref[...] = jnp.dot(s, l3w_ref[...], preferred_element_type=jnp.float32) + l3b_ref[...][None, :]


def _tcd(s, L1w, L1b, L2w, L2b, L3w, L3b):
    return pl.pallas_call(
        _tcd_body,
        out_shape=jax.ShapeDtypeStruct((1, 10), jnp.float32),
    )(s, L1w, L1b, L2w, L2b, L3w, L3b)


_SC_MESH = plsc.VectorSubcoreMesh(core_axis_name="c", subcore_axis_name="s")


@functools.partial(
    pl.kernel,
    out_type=jax.ShapeDtypeStruct((2, NP), jnp.float32),
    mesh=_SC_MESH,
    scratch_types=[
        pltpu.VMEM((NCHS, CHS), jnp.int32),
        pltpu.VMEM((NCHS, CHS), jnp.int32),
        pltpu.VMEM((FANS * CHS,), jnp.float32),
        pltpu.VMEM_SHARED((NP,), jnp.float32),
        [pltpu.SemaphoreType.DMA] * FANS,
    ],
)
def _sc_seg_scalar(vals_hbm, src_hbm, dst_hbm, zvec_hbm, out_hbm,
                   src_v, dst_v, buf_v, acc_sh, sems):
    cid = lax.axis_index("c")
    sid = lax.axis_index("s")
    wid = sid * NC + cid
    pltpu.sync_copy(zvec_hbm, acc_sh.at[pl.ds(sid * NPS, NPS)])
    pltpu.sync_copy(src_hbm.at[wid], src_v)
    pltpu.sync_copy(dst_hbm.at[wid], dst_v)
    plsc.subcore_barrier()

    def body(jj, carry):
        base = jj * FANS
        cps = [
            pltpu.async_copy(vals_hbm.at[src_v.at[base + b]],
                             buf_v.at[pl.ds(b * CHS, CHS)], sems[b])
            for b in range(FANS)
        ]
        for b in range(FANS):
            cps[b].wait()
            pltpu.sync_copy(buf_v.at[pl.ds(b * CHS, CHS)],
                            acc_sh.at[dst_v.at[base + b]], add=True)
        return carry

    lax.fori_loop(0, NCHS // FANS, body, 0)
    plsc.subcore_barrier()
    pltpu.sync_copy(acc_sh.at[pl.ds(sid * NPS, NPS)],
                    out_hbm.at[cid, pl.ds(sid * NPS, NPS)])


@functools.partial(
    pl.kernel,
    out_type=jax.ShapeDtypeStruct((2, NP, D), jnp.float32),
    mesh=_SC_MESH,
    scratch_types=[
        pltpu.VMEM((FANR, CHR), jnp.int32),
        pltpu.VMEM((FANR, CHR), jnp.int32),
        pltpu.VMEM((FANR * CHR, D), jnp.float32),
        pltpu.VMEM_SHARED((NP, D), jnp.float32),
        [pltpu.SemaphoreType.DMA] * FANR,
    ],
)
def _sc_seg_rows(y_hbm, src_hbm, dst_hbm, zrows_hbm, out_hbm,
                 src_v, dst_v, rows_v, acc_sh, sems):
    cid = lax.axis_index("c")
    sid = lax.axis_index("s")
    wid = sid * NC + cid
    pltpu.sync_copy(zrows_hbm, acc_sh.at[pl.ds(sid * NPS, NPS)])
    plsc.subcore_barrier()

    def body(jj, carry):
        base = jj * FANR
        cps = []
        for b in range(FANR):
            pltpu.sync_copy(src_hbm.at[wid, base + b], src_v.at[b])
            pltpu.sync_copy(dst_hbm.at[wid, base + b], dst_v.at[b])
            cps.append(pltpu.async_copy(y_hbm.at[src_v.at[b]],
                                        rows_v.at[pl.ds(b * CHR, CHR)],
                                        sems[b]))
        for b in range(FANR):
            cps[b].wait()
            pltpu.sync_copy(rows_v.at[pl.ds(b * CHR, CHR)],
                            acc_sh.at[dst_v.at[b]], add=True)
        return carry

    lax.fori_loop(0, NCHR // FANR, body, 0)
    plsc.subcore_barrier()
    pltpu.sync_copy(acc_sh.at[pl.ds(sid * NPS, NPS)],
                    out_hbm.at[cid, pl.ds(sid * NPS, NPS)])


def kernel(x, edge_index, batch, W1, b1, Ws1, bs1, W2, b2, Ws2, bs2, W3, b3,
           Ws3, bs3, L1w, L1b, L2w, L2b, L3w, L3b):
    npad = E2 - E
    piota = jnp.arange(npad, dtype=edge_index.dtype)
    pad_src = N + piota % (NP - N)
    pad_dst = piota % NP
    srcp = jnp.concatenate([edge_index[0], pad_src])
    dstp = jnp.concatenate([edge_index[1], pad_dst])
    src3 = jnp.reshape(srcp, (NW, NCHS, CHS))
    dst3 = jnp.reshape(dstp, (NW, NCHS, CHS))
    src2r = jnp.reshape(srcp, (NW, NCHR, CHR))
    dst2r = jnp.reshape(dstp, (NW, NCHR, CHR))
    zvec = jnp.zeros((NPS,), jnp.float32)
    zrows = jnp.zeros((NPS, D), jnp.float32)
    h = jnp.pad(x, ((0, NP - N), (0, 0)))
    a = jnp.pad(jnp.ones((N,), jnp.float32), (0, NP - N))

    ks = []
    kk = N
    for _ in range(3):
        kk = int(math.ceil(0.8 * kk))
        ks.append(kk)

    params = [(W1, b1, Ws1, bs1), (W2, b2, Ws2, bs2), (W3, b3, Ws3, bs3)]
    readouts = []
    for r in range(3):
        W, b, Wsc, bsc = params[r]
        k = ks[r]
        xw = _tcx(h, W)
        degsum2 = _sc_seg_scalar(a, src3, dst3, zvec)
        y, ad, dis2 = _tca(xw, degsum2, a)
        ragg2 = _sc_seg_rows(y, src2r, dst2r, zrows)
        hh, xws, z = _tcb(ragg2, xw, ad, dis2, b, Wsc)
        sagg2 = _sc_seg_scalar(z, src3, dst3, zvec)
        if r < 2:
            h, a, ro = _tcc(k, hh, sagg2, xws, ad, dis2, a, bsc)
            readouts.append(ro)
        else:
            out = _tcc3(k, hh, sagg2, xws, ad, dis2, a, bsc,
                        readouts[0], readouts[1],
                        (L1w, L1b, L2w, L2b, L3w, L3b))
    return out
